# baseline (speedup 1.0000x reference)
import functools
import math

import jax
import jax.numpy as jnp
from jax import lax
from jax.experimental import pallas as pl
from jax.experimental.pallas import tpu as pltpu
from jax.experimental.pallas import tpu_sc as plsc

D_MODEL = 512
SCALE = math.sqrt(float(D_MODEL))
LANES = 16

NUM_CORES = 2
NUM_SUBCORES = 16
NUM_WORKERS = NUM_CORES * NUM_SUBCORES

SEQ_PAD = 56
NBUF = 2
VECS_PER_ROW = D_MODEL // LANES


def _make_emb_kernel(num_seq, seq_len):
    assert num_seq % (NUM_WORKERS * NBUF) == 0
    s_per_w = num_seq // NUM_WORKERS
    n_outer = s_per_w // NBUF

    mesh = plsc.VectorSubcoreMesh(core_axis_name="c", subcore_axis_name="s")

    @functools.partial(
        pl.kernel,
        out_type=jax.ShapeDtypeStruct((num_seq, seq_len, D_MODEL), jnp.float32),
        mesh=mesh,
        scratch_types=[
            pltpu.VMEM((s_per_w * SEQ_PAD,), jnp.int32),
            [pltpu.VMEM((seq_len, D_MODEL), jnp.float32) for _ in range(NBUF)],
            [pltpu.VMEM((seq_len, D_MODEL), jnp.float32) for _ in range(NBUF)],
            [pltpu.SemaphoreType.DMA for _ in range(NBUF)],
            [pltpu.SemaphoreType.DMA for _ in range(NBUF)],
        ],
    )
    def emb(table_hbm, idx_hbm, out_hbm, idx_v, gbufs, obufs, gsems, osems):
        wid = lax.axis_index("s") * NUM_CORES + lax.axis_index("c")
        base = wid * s_per_w
        pltpu.sync_copy(idx_hbm.at[pl.ds(base * SEQ_PAD, s_per_w * SEQ_PAD)], idx_v)

        def start_gather(c, b):
            pltpu.async_copy(
                table_hbm.at[idx_v.at[pl.ds(c * SEQ_PAD, seq_len)]], gbufs[b], gsems[b]
            )

        def gather_wait(b):
            pltpu.make_async_copy(
                table_hbm.at[idx_v.at[pl.ds(0, seq_len)]], gbufs[b], gsems[b]
            ).wait()

        def out_descr(c, b):
            return pltpu.make_async_copy(obufs[b], out_hbm.at[base + c], osems[b])

        def scale(b):
            def scale_row(r, carry):
                for j in range(VECS_PER_ROW):
                    sl = pl.ds(j * LANES, LANES)
                    obufs[b][r, sl] = gbufs[b][r, sl] * SCALE
                return carry

            lax.fori_loop(0, seq_len, scale_row, 0, unroll=False)

        for b in range(NBUF):
            start_gather(b, b)

        for b in range(NBUF):
            gather_wait(b)
            scale(b)
            out_descr(b, b).start()
            start_gather(NBUF + b, b)

        @pl.loop(1, n_outer - 1)
        def outer(o):
            c0 = o * NBUF
            for b in range(NBUF):
                c = c0 + b
                gather_wait(b)
                out_descr(c - NBUF, b).wait()
                scale(b)
                out_descr(c, b).start()
                start_gather(c + NBUF, b)

        for b in range(NBUF):
            c = (n_outer - 1) * NBUF + b
            gather_wait(b)
            out_descr(c - NBUF, b).wait()
            scale(b)
            out_descr(c, b).start()

        for b in range(NBUF):
            c = (n_outer - 1) * NBUF + b
            out_descr(c, b).wait()

    return emb


N_SPLITS = 2


@jax.jit
def kernel(x, table):
    b, s = x.shape
    idx = jnp.pad(x.astype(jnp.int32), ((0, 0), (0, SEQ_PAD - s)))
    emb = _make_emb_kernel(b // N_SPLITS, s)
    step = b // N_SPLITS
    outs = [
        emb(table, idx[k * step : (k + 1) * step].reshape(-1))
        for k in range(N_SPLITS)
    ]
    return jnp.concatenate(outs, axis=0)


# speedup vs baseline: 1.4296x; 1.4296x over previous
import functools
import math

import jax
import jax.numpy as jnp
from jax import lax
from jax.experimental import pallas as pl
from jax.experimental.pallas import tpu as pltpu
from jax.experimental.pallas import tpu_sc as plsc

D_MODEL = 512
SCALE = math.sqrt(float(D_MODEL))
LANES = 16

NUM_CORES = 2
NUM_SUBCORES = 16
NUM_WORKERS = NUM_CORES * NUM_SUBCORES

SEQ_PAD = 56
NBUF = 2
VECS_PER_ROW = D_MODEL // LANES


def _make_emb_kernel(num_seq, seq_len):
    assert num_seq % (NUM_WORKERS * NBUF) == 0
    s_per_w = num_seq // NUM_WORKERS
    n_outer = s_per_w // NBUF
    full = (seq_len // 8) * 8
    rem = seq_len - full
    rem_elems = rem * D_MODEL

    mesh = plsc.VectorSubcoreMesh(core_axis_name="c", subcore_axis_name="s")

    @functools.partial(
        pl.kernel,
        out_type=(
            jax.ShapeDtypeStruct((num_seq, seq_len, D_MODEL), jnp.float32),
            jax.ShapeDtypeStruct((num_seq * rem_elems,), jnp.float32),
        ),
        mesh=mesh,
        compiler_params=pltpu.CompilerParams(use_tc_tiling_on_sc=True),
        scratch_types=[
            pltpu.VMEM((s_per_w * SEQ_PAD,), jnp.int32),
            [pltpu.VMEM((SEQ_PAD, D_MODEL), jnp.float32) for _ in range(NBUF)],
            [pltpu.VMEM((full, D_MODEL), jnp.float32) for _ in range(NBUF)],
            [pltpu.VMEM((rem_elems,), jnp.float32) for _ in range(NBUF)],
            [pltpu.SemaphoreType.DMA for _ in range(NBUF)],
            [pltpu.SemaphoreType.DMA for _ in range(NBUF)],
        ],
    )
    def emb(table_hbm, idx_hbm, out_hbm, rem_hbm, idx_v, gbufs, obufs, rbufs, gsems, osems):
        wid = lax.axis_index("s") * NUM_CORES + lax.axis_index("c")
        base = wid * s_per_w
        pltpu.sync_copy(idx_hbm.at[pl.ds(base * SEQ_PAD, s_per_w * SEQ_PAD)], idx_v)

        def start_gather(c, b):
            pltpu.async_copy(
                table_hbm.at[idx_v.at[pl.ds(c * SEQ_PAD, SEQ_PAD)]], gbufs[b], gsems[b]
            )

        def gather_wait(b):
            pltpu.make_async_copy(
                table_hbm.at[idx_v.at[pl.ds(0, SEQ_PAD)]], gbufs[b], gsems[b]
            ).wait()

        def out_start(c, b):
            pltpu.async_copy(
                obufs[b],
                out_hbm.at[base + c, pl.ds(0, full), :],
                osems[b],
            )
            pltpu.async_copy(
                rbufs[b],
                rem_hbm.at[pl.ds((base + c) * rem_elems, rem_elems)],
                osems[b],
            )

        def out_wait(b):
            pltpu.make_async_copy(
                obufs[b],
                out_hbm.at[base, pl.ds(0, full), :],
                osems[b],
            ).wait()
            pltpu.make_async_copy(
                rbufs[b],
                rem_hbm.at[pl.ds(0, rem_elems)],
                osems[b],
            ).wait()

        def scale(b):
            def scale_row(r, carry):
                for j in range(VECS_PER_ROW):
                    sl = pl.ds(j * LANES, LANES)
                    obufs[b][r, sl] = gbufs[b][r, sl] * SCALE
                return carry

            lax.fori_loop(0, full, scale_row, 0, unroll=False)
            for r in range(rem):
                for j in range(VECS_PER_ROW):
                    rbufs[b][pl.ds(r * D_MODEL + j * LANES, LANES)] = (
                        gbufs[b][full + r, pl.ds(j * LANES, LANES)] * SCALE
                    )

        for b in range(NBUF):
            start_gather(b, b)

        for b in range(NBUF):
            gather_wait(b)
            scale(b)
            out_start(b, b)
            start_gather(NBUF + b, b)

        @pl.loop(1, n_outer - 1)
        def outer(o):
            c0 = o * NBUF
            for b in range(NBUF):
                c = c0 + b
                gather_wait(b)
                out_wait(b)
                scale(b)
                out_start(c, b)
                start_gather(c + NBUF, b)

        for b in range(NBUF):
            c = (n_outer - 1) * NBUF + b
            gather_wait(b)
            out_wait(b)
            scale(b)
            out_start(c, b)

        for b in range(NBUF):
            out_wait(b)

    return emb


@jax.jit
def kernel(x, table):
    b, s = x.shape
    idx = jnp.pad(
        x.astype(jnp.int32), ((0, 0), (0, SEQ_PAD - s)), mode="wrap"
    ).reshape(-1)
    emb = _make_emb_kernel(b, s)
    out, rem_flat = emb(table, idx)
    full = (s // 8) * 8
    rem = rem_flat.reshape(b, s - full, D_MODEL)
    return lax.dynamic_update_slice(out, rem, (0, full, 0))
